# baseline (device time: 12211 ns/iter reference)
import jax
import jax.numpy as jnp
from jax import lax
from jax.experimental import pallas as pl
from jax.experimental.pallas import tpu as pltpu

Z = 4


def kernel(x, dy, gamma):
    m, d = x.shape

    def body(x_ref, dy_ref, gamma_ref, out_ref, mine_ref, comm_ref,
             send_sems, recv_sems):
        my_x = lax.axis_index("x")
        my_y = lax.axis_index("y")
        my_z = lax.axis_index("z")

        barrier_sem = pltpu.get_barrier_semaphore()
        for dz in range(1, Z):
            peer_z = lax.rem(my_z + dz, Z)
            pl.semaphore_signal(
                barrier_sem, inc=1,
                device_id=(my_x, my_y, peer_z),
                device_id_type=pl.DeviceIdType.MESH,
            )

        xv = x_ref[...]
        dyv = dy_ref[...]
        inv_d = jnp.float32(1.0 / d)
        mu = jnp.sum(xv, axis=1, keepdims=True) * inv_d
        xc = xv - mu
        var = jnp.sum(xc * xc, axis=1, keepdims=True) * inv_d
        rstd = lax.rsqrt(var + 1e-5)
        mine_ref[0:1, :] = jnp.sum(dyv * (xc * rstd), axis=0, keepdims=True)
        mine_ref[1:2, :] = jnp.sum(dyv, axis=0, keepdims=True)

        pl.semaphore_wait(barrier_sem, Z - 1)

        rdmas = []
        for dz in range(1, Z):
            peer_z = lax.rem(my_z + dz, Z)
            rdma = pltpu.make_async_remote_copy(
                src_ref=mine_ref,
                dst_ref=comm_ref.at[dz - 1],
                send_sem=send_sems.at[dz - 1],
                recv_sem=recv_sems.at[dz - 1],
                device_id=(my_x, my_y, peer_z),
                device_id_type=pl.DeviceIdType.MESH,
            )
            rdma.start()
            rdmas.append(rdma)

        acc = mine_ref[...]
        for dz in range(1, Z):
            rdmas[dz - 1].wait_recv()
            acc = acc + comm_ref[dz - 1]
        out_ref[...] = acc

        for rdma in rdmas:
            rdma.wait_send()

    return pl.pallas_call(
        body,
        out_shape=jax.ShapeDtypeStruct((2, d), jnp.float32),
        in_specs=[
            pl.BlockSpec(memory_space=pltpu.VMEM),
            pl.BlockSpec(memory_space=pltpu.VMEM),
            pl.BlockSpec(memory_space=pltpu.VMEM),
        ],
        out_specs=pl.BlockSpec(memory_space=pltpu.VMEM),
        scratch_shapes=[
            pltpu.VMEM((2, d), jnp.float32),
            pltpu.VMEM((Z - 1, 2, d), jnp.float32),
            pltpu.SemaphoreType.DMA((Z - 1,)),
            pltpu.SemaphoreType.DMA((Z - 1,)),
        ],
        compiler_params=pltpu.CompilerParams(collective_id=0),
    )(x, dy, gamma)
